# Initial kernel scaffold; baseline (speedup 1.0000x reference)
import functools

import jax
import jax.numpy as jnp
from jax import lax
from jax.experimental import pallas as pl
from jax.experimental.pallas import tpu as pltpu
from jax.experimental.pallas import tpu_sc as plsc

N = 10000
D = 128
H = 128
C = 10
G = 64
K = 16
ALPHA = 0.05
COEF = (1.0 - ALPHA) / K

NPAD = 10240
ET = N + 320000
NW = 32
EB = 128
NBT = 81
EPAD = NW * NBT * EB
RT = NPAD // 16
RB = 128
NSB = RT // RB

_mesh = plsc.VectorSubcoreMesh(core_axis_name="c", subcore_axis_name="s")
_f32 = jnp.float32


def _zero_buf(buf, rows, width):
    zv = jnp.zeros((16,), _f32)

    def body(r, _):
        for cb in range(width // 16):
            buf[r, pl.ds(cb * 16, 16)] = zv
        return 0

    lax.fori_loop(0, rows, body, 0)


@functools.partial(
    pl.kernel,
    out_type=jax.ShapeDtypeStruct((2, NPAD, 16), _f32),
    mesh=_mesh,
    scratch_types=[
        pltpu.VMEM((NBT, EB), jnp.int32),
        pltpu.VMEM((EB, 16), _f32),
        pltpu.VMEM((EB, 16), _f32),
        pltpu.VMEM_SHARED((NPAD, 16), _f32),
    ],
)
def _deg(coli, degp, colbuf, ones16, buf16, dsh):
    c = lax.axis_index("c")
    s = lax.axis_index("s")
    tid = c * 16 + s
    pltpu.sync_copy(coli.at[pl.ds(tid * NBT, NBT)], colbuf)
    ov = jnp.ones((16,), _f32)

    def fill(r, _):
        ones16[r, pl.ds(0, 16)] = ov
        return 0

    lax.fori_loop(0, EB, fill, 0)
    _zero_buf(buf16, EB, 16)
    for sb in range(NSB):
        pltpu.sync_copy(buf16, dsh.at[pl.ds(s * RT + sb * RB, RB)])
    plsc.subcore_barrier()

    def scat(j, _):
        pltpu.sync_copy(ones16, dsh.at[colbuf.at[j]], add=True)
        return 0

    lax.fori_loop(0, NBT, scat, 0)
    plsc.subcore_barrier()
    for sb in range(NSB):
        r0 = s * RT + sb * RB
        pltpu.sync_copy(dsh.at[pl.ds(r0, RB)], buf16)
        pltpu.sync_copy(buf16, degp.at[c, pl.ds(r0, RB)])


def _make_step(first):
    outs = [jax.ShapeDtypeStruct((NPAD, D), _f32)] * (4 if first else 5)

    @functools.partial(
        pl.kernel,
        out_type=tuple(outs),
        mesh=_mesh,
        scratch_types=[
            pltpu.VMEM((NBT, EB), jnp.int32),
            pltpu.VMEM((NBT, EB), jnp.int32),
            pltpu.VMEM((RB, D), _f32),
            pltpu.VMEM((RB, D), _f32),
            pltpu.VMEM((RB, D), _f32),
            pltpu.VMEM((RB, D), _f32),
            pltpu.VMEM((RT,), _f32),
            pltpu.VMEM_SHARED((NPAD, D), _f32),
            pltpu.SemaphoreType.DMA,
            pltpu.SemaphoreType.DMA,
        ],
    )
    def step(*refs):
        if first:
            (x_h, scl_h, rowi, coli, u0_h, u1_h, a_out, b_out,
             rowbuf, colbuf, st0, st1, abuf, bbuf, sbuf, sclbuf,
             wsh, sem0, sem1) = refs
            a_h = b_h = sin_h = sout_h = None
        else:
            (a_h, b_h, sin_h, scl_h, rowi, coli, u0_h, u1_h, a_out, b_out,
             sout_h, rowbuf, colbuf, st0, st1, abuf, bbuf, sbuf, sclbuf,
             wsh, sem0, sem1) = refs
            x_h = None
        c = lax.axis_index("c")
        s = lax.axis_index("s")
        tid = c * 16 + s
        pltpu.sync_copy(rowi.at[pl.ds(tid * NBT, NBT)], rowbuf)
        pltpu.sync_copy(coli.at[pl.ds(tid * NBT, NBT)], colbuf)
        r_base = s * RT
        pltpu.sync_copy(scl_h.at[pl.ds(r_base, RT)], sclbuf)

        def run(u_ref, out_ref, do_s):
            for sb in range(NSB):
                r0 = r_base + sb * RB
                if first:
                    pltpu.sync_copy(x_h.at[pl.ds(r0, RB)], abuf)
                else:
                    pltpu.sync_copy(a_h.at[pl.ds(r0, RB)], abuf)
                    pltpu.sync_copy(b_h.at[pl.ds(r0, RB)], bbuf)

                    def addrow(r, _):
                        for cb in range(D // 16):
                            sl = pl.ds(cb * 16, 16)
                            abuf[r, sl] = abuf[r, sl] + bbuf[r, sl]
                        return 0

                    lax.fori_loop(0, RB, addrow, 0)
                    if do_s:
                        pltpu.sync_copy(sin_h.at[pl.ds(r0, RB)], sbuf)

                        def srow(r, _):
                            for cb in range(D // 16):
                                sl = pl.ds(cb * 16, 16)
                                sbuf[r, sl] = sbuf[r, sl] + abuf[r, sl]
                            return 0

                        lax.fori_loop(0, RB, srow, 0)
                        pltpu.sync_copy(sbuf, sout_h.at[pl.ds(r0, RB)])

                def urow(r, _):
                    sc = sclbuf[sb * RB + r]
                    for cb in range(D // 16):
                        sl = pl.ds(cb * 16, 16)
                        abuf[r, sl] = abuf[r, sl] * sc
                    return 0

                lax.fori_loop(0, RB, urow, 0)
                pltpu.sync_copy(abuf, u_ref.at[pl.ds(r0, RB)])
            _zero_buf(st0, RB, D)
            for sb in range(NSB):
                pltpu.sync_copy(st0, wsh.at[pl.ds(r_base + sb * RB, RB)])
            plsc.subcore_barrier()
            pltpu.async_copy(u_ref.at[rowbuf.at[0]], st0, sem0)

            def edges(jo, _):
                b0 = 2 * jo
                pltpu.make_async_copy(u_ref.at[rowbuf.at[b0]], st0, sem0).wait()
                pltpu.async_copy(u_ref.at[rowbuf.at[b0 + 1]], st1, sem1)
                pltpu.sync_copy(st0, wsh.at[colbuf.at[b0]], add=True)
                pltpu.make_async_copy(
                    u_ref.at[rowbuf.at[b0 + 1]], st1, sem1).wait()
                pltpu.async_copy(u_ref.at[rowbuf.at[b0 + 2]], st0, sem0)
                pltpu.sync_copy(st1, wsh.at[colbuf.at[b0 + 1]], add=True)
                return 0

            lax.fori_loop(0, (NBT - 1) // 2, edges, 0)
            pltpu.make_async_copy(u_ref.at[rowbuf.at[NBT - 1]], st0, sem0).wait()
            pltpu.sync_copy(st0, wsh.at[colbuf.at[NBT - 1]], add=True)
            plsc.subcore_barrier()
            for sb in range(NSB):
                r0 = r_base + sb * RB
                pltpu.sync_copy(wsh.at[pl.ds(r0, RB)], abuf)
                pltpu.sync_copy(abuf, out_ref.at[pl.ds(r0, RB)])

        pl.when(c == 0)(lambda: run(u0_h, a_out, not first))
        pl.when(c == 1)(lambda: run(u1_h, b_out, False))

    return step


_step1 = _make_step(True)
_stepk = _make_step(False)


BN = 256
NBLK = NPAD // BN
_HP = jax.lax.Precision.HIGHEST


def _head_body(x_ref, s_ref, a_ref, b_ref, dis_ref, bat_ref, cw_ref, cb_ref,
               l1w_ref, l1b_ref, l2w_ref, l2b_ref, out_ref, pool_ref, cnt_ref):
    i = pl.program_id(0)

    @pl.when(i == 0)
    def _():
        pool_ref[...] = jnp.zeros_like(pool_ref)
        cnt_ref[...] = jnp.zeros_like(cnt_ref)

    stot = s_ref[...] + a_ref[...] + b_ref[...]
    h = ALPHA * x_ref[...] + COEF * (dis_ref[...] * stot)
    ids = jax.lax.broadcasted_iota(jnp.int32, (BN, G), 1)
    oh = (bat_ref[...] == ids).astype(_f32)
    pool_ref[...] += jax.lax.dot_general(
        oh, h, (((0,), (0,)), ((), ())),
        preferred_element_type=_f32, precision=_HP)
    cnt_ref[...] += jax.lax.dot_general(
        oh, jnp.ones((BN, D), _f32), (((0,), (0,)), ((), ())),
        preferred_element_type=_f32, precision=_HP)

    @pl.when(i == NBLK - 1)
    def _():
        cnt = cnt_ref[...]
        pooled = pool_ref[...] / jnp.maximum(cnt, 1.0)
        nz = jnp.minimum(cnt, 1.0)
        y = jax.lax.dot_general(
            pooled, cw_ref[...], (((1,), (1,)), ((), ())),
            preferred_element_type=_f32, precision=_HP) + cb_ref[0:1, :] * nz
        z = jax.lax.dot_general(
            y, l1w_ref[...], (((1,), (1,)), ((), ())),
            preferred_element_type=_f32, precision=_HP) + l1b_ref[0:1, :]
        z = jnp.maximum(z, 0.0)
        o = jax.lax.dot_general(
            z, l2w_ref[...], (((1,), (1,)), ((), ())),
            preferred_element_type=_f32, precision=_HP) + l2b_ref[0:1, :]
        o10 = o[:, :C]
        m = jnp.max(o10, axis=1, keepdims=True)
        lse = m + jnp.log(jnp.sum(jnp.exp(o10 - m), axis=1, keepdims=True))
        res = o10 - lse
        out_ref[...] = jnp.concatenate(
            [res, jnp.zeros((G, 16 - C), _f32)], axis=1)


def _head(xp, s15, a16, b16, dis2, batp, cw, cb, l1w, l1b, l2w, l2b):
    blk = lambda *shape: pl.BlockSpec(shape, lambda i: (0,) * len(shape))
    row = pl.BlockSpec((BN, D), lambda i: (i, 0))
    nar = pl.BlockSpec((BN, 1), lambda i: (i, 0))
    return pl.pallas_call(
        _head_body,
        grid=(NBLK,),
        in_specs=[row, row, row, row, nar, nar,
                  blk(H, D), blk(8, D), blk(H, H), blk(8, H),
                  blk(16, H), blk(8, 16)],
        out_specs=blk(G, 16),
        out_shape=jax.ShapeDtypeStruct((G, 16), _f32),
        scratch_shapes=[pltpu.VMEM((G, D), _f32), pltpu.VMEM((G, D), _f32)],
        compiler_params=pltpu.CompilerParams(
            dimension_semantics=("arbitrary",)),
    )(xp, s15, a16, b16, dis2, batp, cw, cb, l1w, l1b, l2w, l2b)


def kernel(x, edge_index, batch, conv_W, conv_b, lin1_W, lin1_b, lin2_W,
           lin2_b):
    i32 = jnp.int32
    loops = jnp.arange(N, dtype=i32)
    row = jnp.concatenate([edge_index[0], loops,
                           jnp.zeros((EPAD - ET,), i32)])
    col = jnp.concatenate([edge_index[1], loops,
                           jnp.full((EPAD - ET,), NPAD - 1, i32)])
    rowi = row.reshape(NW * NBT, EB)
    coli = col.reshape(NW * NBT, EB)

    degp = _deg(coli)
    deg = degp[0, :, 0] + degp[1, :, 0]
    dis = jnp.where(deg > 0, jax.lax.rsqrt(deg), 0.0).astype(_f32)
    dinv = jnp.where(deg > 0, 1.0 / deg, 0.0).astype(_f32)

    xp = jnp.pad(x, ((0, NPAD - N), (0, 0)))
    u0, u1, a, b = _step1(xp, dis, rowi, coli)
    s = jnp.zeros((NPAD, D), _f32)
    for _ in range(K - 1):
        u0, u1, a, b, s = _stepk(a, b, s, dinv, rowi, coli)

    batp = jnp.pad(batch, (0, NPAD - N), constant_values=G).reshape(NPAD, 1)
    dis2 = dis.reshape(NPAD, 1)
    cb = jnp.broadcast_to(conv_b.reshape(1, D), (8, D))
    l1b = jnp.broadcast_to(lin1_b.reshape(1, H), (8, H))
    l2w = jnp.pad(lin2_W, ((0, 16 - C), (0, 0)))
    l2b = jnp.broadcast_to(jnp.pad(lin2_b, (0, 16 - C)).reshape(1, 16), (8, 16))
    outp = _head(xp, s, a, b, dis2, batp, conv_W, cb, lin1_W, l1b, l2w, l2b)
    return outp[:, :C]


# speedup vs baseline: 9.7877x; 9.7877x over previous
import functools

import jax
import jax.numpy as jnp
from jax import lax
from jax.experimental import pallas as pl
from jax.experimental.pallas import tpu as pltpu
from jax.experimental.pallas import tpu_sc as plsc

N = 10000
D = 128
H = 128
C = 10
G = 64
K = 16
ALPHA = 0.05
COEF = (1.0 - ALPHA) / K

NPAD = 10240
ET = N + 320000
NW = 32
EB = 128
NBT = 81
EPAD = NW * NBT * EB
NB0 = 41
NB1 = NBT - NB0
NBW = 48
NIR = NW * NBT + 8
RT = NPAD // 16
RB = 128
NSB = RT // RB

_mesh = plsc.VectorSubcoreMesh(core_axis_name="c", subcore_axis_name="s")
_f32 = jnp.float32


def _zero_buf(buf, rows, width):
    zv = jnp.zeros((16,), _f32)

    def body(r, _):
        for cb in range(width // 16):
            buf[r, pl.ds(cb * 16, 16)] = zv
        return 0

    lax.fori_loop(0, rows, body, 0)


def _align8(i):
    return pl.multiple_of((i >> 3) << 3, 8)


@functools.partial(
    pl.kernel,
    out_type=jax.ShapeDtypeStruct((2, NPAD, 16), _f32),
    mesh=_mesh,
    scratch_types=[
        pltpu.VMEM((NBT + 7, EB), jnp.int32),
        pltpu.VMEM((EB, 16), _f32),
        pltpu.VMEM((RB, 16), _f32),
        pltpu.VMEM_SHARED((NPAD, 16), _f32),
    ],
)
def _deg(coli, degp, colbuf, ones16, buf16, dsh):
    c = lax.axis_index("c")
    s = lax.axis_index("s")
    tid = c * 16 + s
    a0 = _align8(tid * NBT)
    off = tid * NBT - a0
    pltpu.sync_copy(coli.at[pl.ds(a0, NBT + 7)], colbuf)
    ov = jnp.ones((16,), _f32)

    def fill(r, _):
        ones16[r, pl.ds(0, 16)] = ov
        return 0

    lax.fori_loop(0, EB, fill, 0)
    _zero_buf(buf16, RB, 16)
    for sb in range(NSB):
        pltpu.sync_copy(buf16, dsh.at[pl.ds(s * RT + sb * RB, RB)])
    plsc.subcore_barrier()

    def scat(j, _):
        pltpu.sync_copy(ones16, dsh.at[colbuf.at[off + j]], add=True)
        return 0

    lax.fori_loop(0, NBT, scat, 0)
    plsc.subcore_barrier()
    for sb in range(NSB):
        r0 = s * RT + sb * RB
        pltpu.sync_copy(dsh.at[pl.ds(r0, RB)], buf16)
        pltpu.sync_copy(buf16, degp.at[c, pl.ds(r0, RB)])


def _make_step(first):
    outs = [jax.ShapeDtypeStruct((NPAD, D), _f32)] * (4 if first else 5)

    @functools.partial(
        pl.kernel,
        out_type=tuple(outs),
        mesh=_mesh,
        scratch_types=[
            pltpu.VMEM((NBW, EB), jnp.int32),
            pltpu.VMEM((NBW, EB), jnp.int32),
            pltpu.VMEM((EB, D), _f32),
            pltpu.VMEM((EB, D), _f32),
            pltpu.VMEM((RT + 16,), _f32),
            pltpu.VMEM_SHARED((NPAD, D), _f32),
            pltpu.SemaphoreType.DMA,
            pltpu.SemaphoreType.DMA,
        ],
    )
    def step(*refs):
        if first:
            (x_h, scl_h, rowi, coli, u0_h, u1_h, a_out, b_out,
             rowbuf, colbuf, st0, st1, sclbuf, wsh, sem0, sem1) = refs
            a_h = b_h = sin_h = sout_h = None
        else:
            (a_h, b_h, sin_h, scl_h, rowi, coli, u0_h, u1_h, a_out, b_out,
             sout_h, rowbuf, colbuf, st0, st1, sclbuf, wsh, sem0, sem1) = refs
            x_h = None
        c = lax.axis_index("c")
        s = lax.axis_index("s")
        tid = c * 16 + s
        r_base = s * RT
        pltpu.sync_copy(scl_h.at[pl.ds(r_base, RT)], sclbuf.at[pl.ds(0, RT)])

        def load_idx(start):
            a0 = _align8(start)
            pltpu.sync_copy(rowi.at[pl.ds(a0, NBW)], rowbuf)
            pltpu.sync_copy(coli.at[pl.ds(a0, NBW)], colbuf)
            return start - a0

        def ring(uc, n, offh):
            pltpu.async_copy(uc.at[rowbuf.at[offh]], st0, sem0)

            def edges(jo, _):
                b0 = offh + 2 * jo
                pltpu.make_async_copy(uc.at[rowbuf.at[b0]], st0, sem0).wait()
                pltpu.async_copy(uc.at[rowbuf.at[b0 + 1]], st1, sem1)
                pltpu.sync_copy(st0, wsh.at[colbuf.at[b0]], add=True)
                pltpu.make_async_copy(
                    uc.at[rowbuf.at[b0 + 1]], st1, sem1).wait()
                pltpu.async_copy(uc.at[rowbuf.at[b0 + 2]], st0, sem0)
                pltpu.sync_copy(st1, wsh.at[colbuf.at[b0 + 1]], add=True)
                return 0

            lax.fori_loop(0, (n - 1) // 2, edges, 0)
            if n % 2 == 1:
                b = offh + n - 1
                pltpu.make_async_copy(uc.at[rowbuf.at[b]], st0, sem0).wait()
                pltpu.sync_copy(st0, wsh.at[colbuf.at[b]], add=True)
            else:
                b = offh + n - 2
                pltpu.make_async_copy(uc.at[rowbuf.at[b]], st0, sem0).wait()
                pltpu.async_copy(uc.at[rowbuf.at[b + 1]], st1, sem1)
                pltpu.sync_copy(st0, wsh.at[colbuf.at[b]], add=True)
                pltpu.make_async_copy(
                    uc.at[rowbuf.at[b + 1]], st1, sem1).wait()
                pltpu.sync_copy(st1, wsh.at[colbuf.at[b + 1]], add=True)

        def run(u_ref, out_ref, do_s):
            for sb in range(NSB):
                r0 = r_base + sb * RB
                if first:
                    pltpu.sync_copy(x_h.at[pl.ds(r0, RB)], st0)
                else:
                    pltpu.sync_copy(a_h.at[pl.ds(r0, RB)], st0)
                    pltpu.sync_copy(b_h.at[pl.ds(r0, RB)], st1)

                    def addrow(r, _):
                        for cb in range(D // 16):
                            sl = pl.ds(cb * 16, 16)
                            st0[r, sl] = st0[r, sl] + st1[r, sl]
                        return 0

                    lax.fori_loop(0, RB, addrow, 0)
                    if do_s:
                        pltpu.sync_copy(sin_h.at[pl.ds(r0, RB)], st1)

                        def srow(r, _):
                            for cb in range(D // 16):
                                sl = pl.ds(cb * 16, 16)
                                st1[r, sl] = st1[r, sl] + st0[r, sl]
                            return 0

                        lax.fori_loop(0, RB, srow, 0)
                        pltpu.sync_copy(st1, sout_h.at[pl.ds(r0, RB)])

                def urow(r, _):
                    sc = sclbuf[pl.ds(sb * RB + r, 16)][0]
                    for cb in range(D // 16):
                        sl = pl.ds(cb * 16, 16)
                        st0[r, sl] = st0[r, sl] * sc
                    return 0

                lax.fori_loop(0, RB, urow, 0)
                pltpu.sync_copy(st0, u_ref.at[pl.ds(r0, RB)])
            _zero_buf(st0, RB, D)
            for sb in range(NSB):
                pltpu.sync_copy(st0, wsh.at[pl.ds(r_base + sb * RB, RB)])
            plsc.subcore_barrier()
            off = load_idx(tid * NBT)
            ring(u_ref, NB0, off)
            off = load_idx(tid * NBT + NB0)
            ring(u_ref, NB1, off)
            plsc.subcore_barrier()
            for sb in range(NSB):
                r0 = r_base + sb * RB
                pltpu.sync_copy(wsh.at[pl.ds(r0, RB)], st0)
                pltpu.sync_copy(st0, out_ref.at[pl.ds(r0, RB)])

        pl.when(c == 0)(lambda: run(u0_h, a_out, not first))
        pl.when(c == 1)(lambda: run(u1_h, b_out, False))

    return step


_step1 = _make_step(True)
_stepk = _make_step(False)


BN = 256
NBLK = NPAD // BN
_HP = jax.lax.Precision.HIGHEST


def _head_body(x_ref, s_ref, a_ref, b_ref, dis_ref, bat_ref, cw_ref, cb_ref,
               l1w_ref, l1b_ref, l2w_ref, l2b_ref, out_ref, pool_ref, cnt_ref):
    i = pl.program_id(0)

    @pl.when(i == 0)
    def _():
        pool_ref[...] = jnp.zeros_like(pool_ref)
        cnt_ref[...] = jnp.zeros_like(cnt_ref)

    stot = s_ref[...] + a_ref[...] + b_ref[...]
    h = ALPHA * x_ref[...] + COEF * (dis_ref[...] * stot)
    ids = jax.lax.broadcasted_iota(jnp.int32, (BN, G), 1)
    oh = (bat_ref[...] == ids).astype(_f32)
    pool_ref[...] += jax.lax.dot_general(
        oh, h, (((0,), (0,)), ((), ())),
        preferred_element_type=_f32, precision=_HP)
    cnt_ref[...] += jax.lax.dot_general(
        oh, jnp.ones((BN, D), _f32), (((0,), (0,)), ((), ())),
        preferred_element_type=_f32, precision=_HP)

    @pl.when(i == NBLK - 1)
    def _():
        cnt = cnt_ref[...]
        pooled = pool_ref[...] / jnp.maximum(cnt, 1.0)
        nz = jnp.minimum(cnt, 1.0)
        y = jax.lax.dot_general(
            pooled, cw_ref[...], (((1,), (1,)), ((), ())),
            preferred_element_type=_f32, precision=_HP) + cb_ref[0:1, :] * nz
        z = jax.lax.dot_general(
            y, l1w_ref[...], (((1,), (1,)), ((), ())),
            preferred_element_type=_f32, precision=_HP) + l1b_ref[0:1, :]
        z = jnp.maximum(z, 0.0)
        o = jax.lax.dot_general(
            z, l2w_ref[...], (((1,), (1,)), ((), ())),
            preferred_element_type=_f32, precision=_HP) + l2b_ref[0:1, :]
        o10 = o[:, :C]
        m = jnp.max(o10, axis=1, keepdims=True)
        lse = m + jnp.log(jnp.sum(jnp.exp(o10 - m), axis=1, keepdims=True))
        res = o10 - lse
        out_ref[...] = jnp.concatenate(
            [res, jnp.zeros((G, 16 - C), _f32)], axis=1)


def _head(xp, s15, a16, b16, dis2, batp, cw, cb, l1w, l1b, l2w, l2b):
    blk = lambda *shape: pl.BlockSpec(shape, lambda i: (0,) * len(shape))
    row = pl.BlockSpec((BN, D), lambda i: (i, 0))
    nar = pl.BlockSpec((BN, 1), lambda i: (i, 0))
    return pl.pallas_call(
        _head_body,
        grid=(NBLK,),
        in_specs=[row, row, row, row, nar, nar,
                  blk(H, D), blk(8, D), blk(H, H), blk(8, H),
                  blk(16, H), blk(8, 16)],
        out_specs=blk(G, 16),
        out_shape=jax.ShapeDtypeStruct((G, 16), _f32),
        scratch_shapes=[pltpu.VMEM((G, D), _f32), pltpu.VMEM((G, D), _f32)],
        compiler_params=pltpu.CompilerParams(
            dimension_semantics=("arbitrary",)),
    )(xp, s15, a16, b16, dis2, batp, cw, cb, l1w, l1b, l2w, l2b)


def kernel(x, edge_index, batch, conv_W, conv_b, lin1_W, lin1_b, lin2_W,
           lin2_b):
    i32 = jnp.int32
    loops = jnp.arange(N, dtype=i32)
    pad_e = NIR * EB - ET
    row = jnp.concatenate([edge_index[0], loops, jnp.zeros((pad_e,), i32)])
    col = jnp.concatenate([edge_index[1], loops,
                           jnp.full((pad_e,), NPAD - 1, i32)])
    rowi = row.reshape(NIR, EB)
    coli = col.reshape(NIR, EB)

    degp = _deg(coli)
    deg = degp[0, :, 0] + degp[1, :, 0]
    dis = jnp.where(deg > 0, jax.lax.rsqrt(deg), 0.0).astype(_f32)
    dinv = jnp.where(deg > 0, 1.0 / deg, 0.0).astype(_f32)

    xp = jnp.pad(x, ((0, NPAD - N), (0, 0)))
    u0, u1, a, b = _step1(xp, dis, rowi, coli)
    s = jnp.zeros((NPAD, D), _f32)
    for _ in range(K - 1):
        u0, u1, a, b, s = _stepk(a, b, s, dinv, rowi, coli)

    batp = jnp.pad(batch, (0, NPAD - N), constant_values=G).reshape(NPAD, 1)
    dis2 = dis.reshape(NPAD, 1)
    cb = jnp.broadcast_to(conv_b.reshape(1, D), (8, D))
    l1b = jnp.broadcast_to(lin1_b.reshape(1, H), (8, H))
    l2w = jnp.pad(lin2_W, ((0, 16 - C), (0, 0)))
    l2b = jnp.broadcast_to(jnp.pad(lin2_b, (0, 16 - C)).reshape(1, 16), (8, 16))
    outp = _head(xp, s, a, b, dis2, batp, conv_W, cb, lin1_W, l1b, l2w, l2b)
    return outp[:, :C]
